# baseline (speedup 1.0000x reference)
import functools
import math

import jax
import jax.numpy as jnp
from jax import lax
from jax.experimental import pallas as pl
from jax.experimental.pallas import tpu as pltpu
from jax.experimental.pallas import tpu_sc as plsc

B = 128
N = 8193
K = 16
EMB = 2
BR = 32
NEG = -1e9
CHUNK_IN = 256
NCHUNK = 64
INV_SQRT2 = 1.0 / math.sqrt(2.0)


def _sc_gather_build():
    mesh = plsc.VectorSubcoreMesh(core_axis_name="c", subcore_axis_name="s")

    @functools.partial(
        pl.kernel,
        out_type=[jax.ShapeDtypeStruct((B, EMB), jnp.float32),
                  jax.ShapeDtypeStruct((B, K), jnp.int32)],
        mesh=mesh,
        scratch_types=[pltpu.VMEM((16,), jnp.int32),
                       pltpu.VMEM((16,), jnp.int32),
                       pltpu.VMEM((16, EMB), jnp.float32),
                       pltpu.VMEM((16, K), jnp.int32),
                       pltpu.SemaphoreType.DMA],
        compiler_params=pltpu.CompilerParams(use_tc_tiling_on_sc=False),
    )
    def sc_gather(psi2d, knn2d, cur_hbm, psi_out, knn_out,
                  cur_v, idx_rows, psi_rows, knn_rows, sem):
        wid = lax.axis_index("s") * 2 + lax.axis_index("c")

        @pl.when(wid < 8)
        def _():
            base = wid * 16
            pltpu.sync_copy(cur_hbm.at[pl.ds(base, 16)], cur_v)
            lanes = lax.iota(jnp.int32, 16)
            rows = (base + lanes) * N + cur_v[...]
            idx_rows[...] = rows
            pltpu.async_copy(psi2d.at[idx_rows], psi_rows, sem).wait()
            pltpu.async_copy(knn2d.at[idx_rows], knn_rows, sem).wait()
            pltpu.sync_copy(psi_rows, psi_out.at[pl.ds(base, 16)])
            pltpu.sync_copy(knn_rows, knn_out.at[pl.ds(base, 16)])

    return sc_gather


_sc_gather = _sc_gather_build()


def _tc_body(psi_ref, dem_ref, vis_ref, cap_ref, used_ref, cur_ref,
             psic_ref, knn_ref, w_ref, scal_ref,
             out_ref, mask_ref, attn_scr):
    f32 = jnp.float32
    cap = cap_ref[...]
    used = used_ref[...]
    remaining = cap - used
    cur = cur_ref[...]
    at_depot = cur == 0
    psi_c = jnp.where(at_depot, 0.0, psic_ref[...])
    cap_frac = remaining / jnp.maximum(cap, 1e-9)
    step_frac = scal_ref[0, 0]
    lam = scal_ref[0, 1]
    c0 = psi_c[:, 0:1]
    c1 = psi_c[:, 1:2]
    q = []
    for e in range(EMB):
        pre = (c0 * w_ref[0, e] + c1 * w_ref[1, e]
               + cap_frac * w_ref[2, e] + step_frac * w_ref[3, e]
               + w_ref[4, e])
        q.append(jnp.tanh(pre) * INV_SQRT2)

    lane = lax.broadcasted_iota(jnp.int32, (BR, CHUNK_IN), 1)
    qpat = jnp.where((lane % 2) == 0, q[0], q[1])
    jrow = lax.broadcasted_iota(jnp.int32, (CHUNK_IN, 128), 0)
    ncol = lax.broadcasted_iota(jnp.int32, (CHUNK_IN, 128), 1)
    pmat = (jrow // 2 == ncol).astype(f32)
    for c in range(NCHUNK):
        x = psi_ref[:, c * CHUNK_IN:(c + 1) * CHUNK_IN]
        attn_scr[:, c * 128:(c + 1) * 128] = jnp.dot(
            x * qpat, pmat, preferred_element_type=f32)
    x0 = psi_ref[:, 2 * N - 2:2 * N - 1]
    x1 = psi_ref[:, 2 * N - 1:2 * N]
    attn_scr[:, N - 1:N] = x0 * q[0] + x1 * q[1]

    attn_t = 10.0 * jnp.tanh(attn_scr[:, 0:N])

    i32 = jnp.int32
    col = lax.broadcasted_iota(i32, (BR, N), 1)
    mrow = ((vis_ref[...] != 0) | (dem_ref[...] > remaining)).astype(i32)
    cust_un = jnp.where(col >= 1, 1 - mrow, 0)
    has_cust = jnp.max(cust_un, axis=1, keepdims=True) > 0
    m0 = jnp.where(at_depot & has_cust, 1, 0)
    full_i = jnp.where(col == 0, m0, mrow)
    acc = col == knn_ref[:, 0:1]
    for k in range(1, K):
        acc = acc | (col == knn_ref[:, k:k + 1])
    hybrid = (1.0 - lam) * attn_t + jnp.where(acc, lam, 0.0)
    scores = jnp.where(full_i > 0, NEG, hybrid)
    sumexp = jnp.sum(jnp.exp(scores), axis=1, keepdims=True)
    out_ref[...] = scores - jnp.log(sumexp)
    mask_ref[...] = full_i


def _tc_dense(psi_flat, demands, visited, cap2, used2, cur2,
              psi_curr, knn_cur, wmat, scal):
    rb = lambda i: (i, 0)
    return pl.pallas_call(
        _tc_body,
        grid=(B // BR,),
        in_specs=[
            pl.BlockSpec((BR, 2 * N), rb),
            pl.BlockSpec((BR, N), rb),
            pl.BlockSpec((BR, N), rb),
            pl.BlockSpec((BR, 1), rb),
            pl.BlockSpec((BR, 1), rb),
            pl.BlockSpec((BR, 1), rb),
            pl.BlockSpec((BR, EMB), rb),
            pl.BlockSpec((BR, K), rb),
            pl.BlockSpec(memory_space=pltpu.SMEM),
            pl.BlockSpec(memory_space=pltpu.SMEM),
        ],
        out_specs=[pl.BlockSpec((BR, N), rb), pl.BlockSpec((BR, N), rb)],
        out_shape=[jax.ShapeDtypeStruct((B, N), jnp.float32),
                   jax.ShapeDtypeStruct((B, N), jnp.int32)],
        scratch_shapes=[pltpu.VMEM((BR, 8256), jnp.float32)],
        compiler_params=pltpu.CompilerParams(
            dimension_semantics=("arbitrary",)),
    )(psi_flat, demands, visited, cap2, used2, cur2, psi_curr, knn_cur,
      wmat, scal)


def kernel(psi_prime, knn_indices, visited, demands, capacity, used_capacity,
           current_node, step, n_customers, W_ctx, b_ctx, lam):
    f32 = jnp.float32
    psi_flat = psi_prime.reshape(B, N * EMB)
    psi2d = psi_prime.reshape(B * N, EMB)
    knn2d = knn_indices.reshape(B * N, K)
    cur = current_node.astype(jnp.int32)

    psi_curr, knn_cur = _sc_gather(psi2d, knn2d, cur)

    step_f = jnp.asarray(step, f32)
    denom = jnp.maximum(jnp.asarray(n_customers, f32), 1.0)
    scal = jnp.stack([step_f / denom, jnp.asarray(lam, f32)]).reshape(1, 2)
    wmat = jnp.concatenate(
        [W_ctx.astype(f32), b_ctx.reshape(1, EMB).astype(f32)], axis=0)
    cap2 = capacity.reshape(B, 1)
    used2 = used_capacity.reshape(B, 1)
    cur2 = cur.reshape(B, 1)

    vis_u8 = visited.view(jnp.uint8)
    log_probs, mask_i32 = _tc_dense(psi_flat, demands, vis_u8, cap2, used2,
                                    cur2, psi_curr, knn_cur, wmat, scal)
    return log_probs, mask_i32.astype(jnp.bool_)


# speedup vs baseline: 56.9149x; 56.9149x over previous
import functools
import math

import jax
import jax.numpy as jnp
from jax import lax
from jax.experimental import pallas as pl
from jax.experimental.pallas import tpu as pltpu
from jax.experimental.pallas import tpu_sc as plsc

B = 128
N = 8193
K = 16
EMB = 2
NEG = -1e9
INV_SQRT2 = 1.0 / math.sqrt(2.0)


def _sc_gather_build():
    mesh = plsc.VectorSubcoreMesh(core_axis_name="c", subcore_axis_name="s")

    @functools.partial(
        pl.kernel,
        out_type=jax.ShapeDtypeStruct((B * K,), jnp.int32),
        mesh=mesh,
        scratch_types=[pltpu.VMEM((16,), jnp.int32),
                       pltpu.VMEM((256,), jnp.int32),
                       pltpu.VMEM((256,), jnp.int32),
                       pltpu.SemaphoreType.DMA],
        compiler_params=pltpu.CompilerParams(use_tc_tiling_on_sc=False),
    )
    def sc_gather(knn1d, cur_hbm, out1d, cur_v, idx_v, val_v, sem):
        wid = lax.axis_index("s") * 2 + lax.axis_index("c")

        @pl.when(wid < 8)
        def _():
            base = wid * 16
            pltpu.sync_copy(cur_hbm.at[pl.ds(base, 16)], cur_v)
            lanes = lax.iota(jnp.int32, 16)
            flat_b = base + lanes
            row0 = cur_v[...] * (K * B) + flat_b
            for k in range(K):
                idx_v[pl.ds(k * 16, 16)] = row0 + k * B
            pltpu.async_copy(knn1d.at[idx_v], val_v, sem).wait()
            pltpu.sync_copy(val_v, out1d.at[pl.ds(wid * 256, 256)])

    return sc_gather


_sc_gather = _sc_gather_build()


def _tc_body(psi0_ref, psi1_ref, dem_ref, vis_ref, knn_ref, cur_ref,
             cap_ref, used_ref, w_ref, scal_ref, out_ref, mask_ref):
    i32 = jnp.int32
    cap = cap_ref[...]
    rem = cap - used_ref[...]
    cur = cur_ref[...]
    at_depot = cur == 0
    niota = lax.broadcasted_iota(i32, (N, 1), 0)
    onehot = jnp.where(niota == cur, 1.0, 0.0)
    p0 = psi0_ref[...]
    p1 = psi1_ref[...]
    c0 = jnp.sum(p0 * onehot, axis=0, keepdims=True)
    c1 = jnp.sum(p1 * onehot, axis=0, keepdims=True)
    c0 = jnp.where(at_depot, 0.0, c0)
    c1 = jnp.where(at_depot, 0.0, c1)
    cap_frac = rem / jnp.maximum(cap, 1e-9)
    step_frac = scal_ref[0, 0]
    lam = scal_ref[0, 1]
    q = []
    for e in range(EMB):
        pre = (c0 * w_ref[0, e] + c1 * w_ref[1, e]
               + cap_frac * w_ref[2, e] + step_frac * w_ref[3, e]
               + w_ref[4, e])
        q.append(jnp.tanh(pre) * INV_SQRT2)

    attn = p0 * q[0] + p1 * q[1]
    attn_t = 10.0 * jnp.tanh(attn)

    mrow = ((vis_ref[...] != 0) | (dem_ref[...] > rem)).astype(i32)
    cust = jnp.where(niota >= 1, 1 - mrow, 0)
    has_cust = jnp.max(cust, axis=0, keepdims=True) > 0
    m0 = jnp.where(at_depot & has_cust, 1, 0)
    full = jnp.where(niota == 0, m0, mrow)

    kv = knn_ref[...]
    acc = niota == kv[0:1, :]
    for k in range(1, K):
        acc = acc | (niota == kv[k:k + 1, :])
    hybrid = (1.0 - lam) * attn_t + jnp.where(acc, lam, 0.0)
    scores = jnp.where(full > 0, NEG, hybrid)
    sumexp = jnp.sum(jnp.exp(scores), axis=0, keepdims=True)
    out_ref[...] = scores - jnp.log(sumexp)
    mask_ref[...] = full


def _tc_dense(psi0, psi1, dem_t, vis_t, knn16, cur_r, cap_r, used_r,
              wmat, scal):
    vm = pl.BlockSpec(memory_space=pltpu.VMEM)
    sm = pl.BlockSpec(memory_space=pltpu.SMEM)
    return pl.pallas_call(
        _tc_body,
        in_specs=[vm, vm, vm, vm, vm, vm, vm, vm, sm, sm],
        out_specs=[vm, vm],
        out_shape=[jax.ShapeDtypeStruct((N, B), jnp.float32),
                   jax.ShapeDtypeStruct((N, B), jnp.int32)],
    )(psi0, psi1, dem_t, vis_t, knn16, cur_r, cap_r, used_r, wmat, scal)


def kernel(psi_prime, knn_indices, visited, demands, capacity, used_capacity,
           current_node, step, n_customers, W_ctx, b_ctx, lam):
    f32 = jnp.float32
    cur = current_node.astype(jnp.int32)

    knn1d = jnp.transpose(knn_indices, (1, 2, 0)).reshape(-1)
    gath = _sc_gather(knn1d, cur)
    knn16 = jnp.transpose(gath.reshape(8, K, 16), (1, 0, 2)).reshape(K, B)

    psi0 = psi_prime[:, :, 0].T
    psi1 = psi_prime[:, :, 1].T
    dem_t = demands.T
    vis_t = visited.T.astype(jnp.uint8)
    cur_r = cur.reshape(1, B)
    cap_r = capacity.reshape(1, B)
    used_r = used_capacity.reshape(1, B)

    step_f = jnp.asarray(step, f32)
    denom = jnp.maximum(jnp.asarray(n_customers, f32), 1.0)
    scal = jnp.stack([step_f / denom, jnp.asarray(lam, f32)]).reshape(1, 2)
    wmat = jnp.concatenate(
        [W_ctx.astype(f32), b_ctx.reshape(1, EMB).astype(f32)], axis=0)

    out_t, mask_i32 = _tc_dense(psi0, psi1, dem_t, vis_t, knn16,
                                cur_r, cap_r, used_r, wmat, scal)
    return out_t.T, mask_i32.astype(jnp.bool_).T


# speedup vs baseline: 62.3042x; 1.0947x over previous
import functools
import math

import jax
import jax.numpy as jnp
from jax import lax
from jax.experimental import pallas as pl
from jax.experimental.pallas import tpu as pltpu
from jax.experimental.pallas import tpu_sc as plsc

B = 128
N = 8193
K = 16
EMB = 2
NEG = -1e9
INV_SQRT2 = 1.0 / math.sqrt(2.0)


def _sc_gather_build():
    mesh = plsc.VectorSubcoreMesh(core_axis_name="c", subcore_axis_name="s")

    @functools.partial(
        pl.kernel,
        out_type=jax.ShapeDtypeStruct((B * K,), jnp.int32),
        mesh=mesh,
        scratch_types=[pltpu.VMEM((16,), jnp.int32),
                       pltpu.VMEM((256,), jnp.int32),
                       pltpu.VMEM((256,), jnp.int32),
                       pltpu.SemaphoreType.DMA],
        compiler_params=pltpu.CompilerParams(use_tc_tiling_on_sc=False),
    )
    def sc_gather(knn1d, cur_hbm, out1d, cur_v, idx_v, val_v, sem):
        wid = lax.axis_index("s") * 2 + lax.axis_index("c")

        @pl.when(wid < 8)
        def _():
            base = wid * 16
            pltpu.sync_copy(cur_hbm.at[pl.ds(base, 16)], cur_v)
            lanes = lax.iota(jnp.int32, 16)
            flat_b = base + lanes
            row0 = cur_v[...] * (K * B) + flat_b
            for k in range(K):
                idx_v[pl.ds(k * 16, 16)] = row0 + k * B
            pltpu.async_copy(knn1d.at[idx_v], val_v, sem).wait()
            pltpu.sync_copy(val_v, out1d.at[pl.ds(wid * 256, 256)])

    return sc_gather


_sc_gather = _sc_gather_build()


NM = N - 1


def _tc_body(psi_ref, tail_ref, dem_ref, vis_ref, knn_ref, cur_ref,
             cap_ref, used_ref, w_ref, scal_ref, out_ref, mask_ref):
    i32 = jnp.int32
    cap = cap_ref[...]
    rem = cap - used_ref[...]
    cur = cur_ref[...]
    at_depot = cur == 0
    tail = tail_ref[...]
    t0 = tail[0:1, :]
    t1 = tail[1:2, :]

    riota = lax.broadcasted_iota(i32, (2 * NM, 1), 0)
    pv = psi_ref[...]
    cur2 = cur * 2
    c0 = jnp.sum(jnp.where(riota == cur2, pv, 0.0), axis=0, keepdims=True)
    c1 = jnp.sum(jnp.where(riota == cur2 + 1, pv, 0.0), axis=0, keepdims=True)
    c0 = c0 + jnp.where(cur == NM, t0, 0.0)
    c1 = c1 + jnp.where(cur == NM, t1, 0.0)
    c0 = jnp.where(at_depot, 0.0, c0)
    c1 = jnp.where(at_depot, 0.0, c1)
    cap_frac = rem / jnp.maximum(cap, 1e-9)
    step_frac = scal_ref[0, 0]
    lam = scal_ref[0, 1]
    q = []
    for e in range(EMB):
        pre = (c0 * w_ref[0, e] + c1 * w_ref[1, e]
               + cap_frac * w_ref[2, e] + step_frac * w_ref[3, e]
               + w_ref[4, e])
        q.append(jnp.tanh(pre) * INV_SQRT2)

    p0 = psi_ref[0::2, :]
    p1 = psi_ref[1::2, :]
    attn = p0 * q[0] + p1 * q[1]
    attn_t = 10.0 * jnp.tanh(attn)
    attn_tail = 10.0 * jnp.tanh(t0 * q[0] + t1 * q[1])

    niota = lax.broadcasted_iota(i32, (N, 1), 0)
    mrow = ((vis_ref[...] != 0) | (dem_ref[...] > rem)).astype(i32)
    cust = jnp.where(niota >= 1, 1 - mrow, 0)
    has_cust = jnp.max(cust, axis=0, keepdims=True) > 0
    m0 = jnp.where(at_depot & has_cust, 1, 0)
    full = jnp.where(niota == 0, m0, mrow)

    kv = knn_ref[...]
    miota = lax.broadcasted_iota(i32, (NM, 1), 0)
    acc = miota == kv[0:1, :]
    acc_t = kv[0:1, :] == NM
    for k in range(1, K):
        acc = acc | (miota == kv[k:k + 1, :])
        acc_t = acc_t | (kv[k:k + 1, :] == NM)
    hybrid = (1.0 - lam) * attn_t + jnp.where(acc, lam, 0.0)
    hybrid_t = (1.0 - lam) * attn_tail + jnp.where(acc_t, lam, 0.0)
    scores = jnp.where(full[0:NM, :] > 0, NEG, hybrid)
    scores_t = jnp.where(full[NM:N, :] > 0, NEG, hybrid_t)
    sumexp = (jnp.sum(jnp.exp(scores), axis=0, keepdims=True)
              + jnp.exp(scores_t))
    lse = jnp.log(sumexp)
    out_ref[0:NM, :] = scores - lse
    out_ref[NM:N, :] = scores_t - lse
    mask_ref[...] = full


def _tc_dense(psi_int, tail, dem_t, vis_t, knn16, cur_r, cap_r, used_r,
              wmat, scal):
    vm = pl.BlockSpec(memory_space=pltpu.VMEM)
    sm = pl.BlockSpec(memory_space=pltpu.SMEM)
    return pl.pallas_call(
        _tc_body,
        in_specs=[vm, vm, vm, vm, vm, vm, vm, vm, sm, sm],
        out_specs=[vm, vm],
        out_shape=[jax.ShapeDtypeStruct((N, B), jnp.float32),
                   jax.ShapeDtypeStruct((N, B), jnp.int32)],
    )(psi_int, tail, dem_t, vis_t, knn16, cur_r, cap_r, used_r, wmat, scal)


def kernel(psi_prime, knn_indices, visited, demands, capacity, used_capacity,
           current_node, step, n_customers, W_ctx, b_ctx, lam):
    f32 = jnp.float32
    cur = current_node.astype(jnp.int32)

    knn1d = jnp.transpose(knn_indices, (1, 2, 0)).reshape(-1)
    gath = _sc_gather(knn1d, cur)
    knn16 = jnp.transpose(gath.reshape(8, K, 16), (1, 0, 2)).reshape(K, B)

    p3 = jnp.transpose(psi_prime, (1, 2, 0))
    psi_int = p3[:NM].reshape(2 * NM, B)
    tail = p3[NM]
    dem_t = demands.T
    vis_t = visited.T.astype(jnp.uint8)
    cur_r = cur.reshape(1, B)
    cap_r = capacity.reshape(1, B)
    used_r = used_capacity.reshape(1, B)

    step_f = jnp.asarray(step, f32)
    denom = jnp.maximum(jnp.asarray(n_customers, f32), 1.0)
    scal = jnp.stack([step_f / denom, jnp.asarray(lam, f32)]).reshape(1, 2)
    wmat = jnp.concatenate(
        [W_ctx.astype(f32), b_ctx.reshape(1, EMB).astype(f32)], axis=0)

    out_t, mask_i32 = _tc_dense(psi_int, tail, dem_t, vis_t, knn16,
                                cur_r, cap_r, used_r, wmat, scal)
    return out_t.T, mask_i32.astype(jnp.bool_).T
